# baseline (speedup 1.0000x reference)
import jax
import jax.numpy as jnp
from jax import lax
from jax.experimental import pallas as pl
from jax.experimental.pallas import tpu as pltpu
from jax.experimental.pallas import tpu_sc as plsc

NB = 12
B = 64
K = 1024
G = 128
NC = 2
NS = 16
NW = NC * NS
BPW = B // NW
L = 16
GROUPS = K // L

_LN2 = 0.6931471805599453
_INV_DELTA = float(NB) / 3.141592653589793

_C = (2.5546730196161803e-07, 0.999967080943859, -0.49928504912250304,
      0.32722571497347896, -0.22316586411879943, 0.13083342798333364,
      -0.05243753706703084, 0.010009289617861138)


def _log_f32(x):
    xi = plsc.bitcast(x, jnp.int32)
    e = (xi >> 23) - 127
    f = plsc.bitcast((xi & 0x007FFFFF) | 0x3F800000, jnp.float32) - 1.0
    f2 = f * f
    f4 = f2 * f2
    q0 = (_C[0] + _C[1] * f) + (_C[2] + _C[3] * f) * f2
    q1 = (_C[4] + _C[5] * f) + (_C[6] + _C[7] * f) * f2
    return e.astype(jnp.float32) * _LN2 + (q0 + q1 * f4)


def _tree(fn, xs):
    xs = list(xs)
    while len(xs) > 1:
        nxt = [fn(xs[i], xs[i + 1]) for i in range(0, len(xs) - 1, 2)]
        if len(xs) % 2:
            nxt.append(xs[-1])
        xs = nxt
    return xs[0]


UNROLL = 8


_KH = K // 2


def _sc_body(scores_hbm, resid_hbm, cls_hbm, rlab_hbm, oa_hbm, obj_hbm,
             out_hbm, scores_v0, resid_v0, scores_v1, resid_v1,
             cls_v, rlab_v, oa_v, obj_v, stage_v, sem0, sem0b, sem1):
    wid = lax.axis_index("s") * NC + lax.axis_index("c")
    iota16 = lax.iota(jnp.int32, L)
    b0 = wid * BPW

    cp0s = pltpu.async_copy(scores_hbm.at[:, b0, pl.ds(0, _KH)],
                            scores_v0.at[:, pl.ds(0, _KH)], sem0)
    cp0r = pltpu.async_copy(resid_hbm.at[:, b0, pl.ds(0, _KH)],
                            resid_v0.at[:, pl.ds(0, _KH)], sem0)
    pltpu.sync_copy(cls_hbm.at[pl.ds(b0, BPW)], cls_v)
    pltpu.sync_copy(rlab_hbm.at[pl.ds(b0, BPW)], rlab_v)
    pltpu.sync_copy(oa_hbm.at[pl.ds(b0, BPW)], oa_v)
    pltpu.sync_copy(obj_hbm.at[pl.ds(b0, BPW)], obj_v)
    cp0sb = pltpu.async_copy(scores_hbm.at[:, b0, pl.ds(_KH, _KH)],
                             scores_v0.at[:, pl.ds(_KH, _KH)], sem0b)
    cp0rb = pltpu.async_copy(resid_hbm.at[:, b0, pl.ds(_KH, _KH)],
                             resid_v0.at[:, pl.ds(_KH, _KH)], sem0b)
    cp1s = pltpu.async_copy(scores_hbm.at[:, b0 + 1], scores_v1, sem1)
    cp1r = pltpu.async_copy(resid_hbm.at[:, b0 + 1], resid_v1, sem1)

    acc = (jnp.zeros((L,), jnp.float32),
           jnp.zeros((L,), jnp.float32),
           jnp.zeros((L,), jnp.float32))

    waits = [(cp0s, cp0r), (cp0sb, cp0rb), (cp1s, cp1r), None]
    spans = [(0, GROUPS // 2), (GROUPS // 2, GROUPS), (0, GROUPS // 2),
             (GROUPS // 2, GROUPS)]
    for phase in range(4):
        b_local = phase // 2
        scores_v = (scores_v0, scores_v1)[b_local]
        resid_v = (resid_v0, resid_v1)[b_local]
        if waits[phase] is not None:
            waits[phase][0].wait()
            waits[phase][1].wait()
        lo, hi = spans[phase]

        @plsc.parallel_loop(lo, hi, 1, unroll=UNROLL, carry=acc)
        def acc(g, carry):
            acc_ce, acc_hu, acc_obj = carry
            base = g * L
            oa = oa_v[b_local, pl.ds(base, L)]
            obj = obj_v[b_local, pl.ds(base, L)].astype(jnp.float32)
            hcl = plsc.load_gather(cls_v, [jnp.full((L,), b_local, jnp.int32), oa])
            rows = base + iota16
            svals = [scores_v[j, pl.ds(base, L)] for j in range(NB)]
            m = _tree(jnp.maximum, svals)
            se = _tree(jnp.add, [jnp.exp(s - m) for s in svals])
            lse = _log_f32(se) + m
            s_h = plsc.load_gather(scores_v, [hcl, rows])
            ce = lse - s_h
            hrl = plsc.load_gather(
                rlab_v, [jnp.full((L,), b_local, jnp.int32), oa]) * _INV_DELTA
            rn = plsc.load_gather(resid_v, [hcl, rows])
            err = rn - hrl
            ae = jnp.abs(err)
            q = jnp.minimum(ae, 1.0)
            hub = 0.5 * q * q + (ae - q)
            return (acc_ce + ce * obj, acc_hu + hub * obj, acc_obj + obj)

    stage_v[pl.ds(0, L)] = acc[0]
    stage_v[pl.ds(L, L)] = acc[1]
    stage_v[pl.ds(2 * L, L)] = acc[2]
    pltpu.sync_copy(stage_v, out_hbm.at[wid])


@jax.jit
def kernel(heading_class_label, heading_scores, heading_residual_label,
           heading_residuals_normalized, object_assignment, objectness_label):
    scores_t = jnp.transpose(heading_scores, (2, 0, 1))
    resid_t = jnp.transpose(heading_residuals_normalized, (2, 0, 1))

    mesh = plsc.VectorSubcoreMesh(core_axis_name="c", subcore_axis_name="s",
                                  num_cores=NC, num_subcores=NS)
    partials = pl.kernel(
        _sc_body,
        out_type=jax.ShapeDtypeStruct((NW, 3 * L), jnp.float32),
        mesh=mesh,
        compiler_params=pltpu.CompilerParams(needs_layout_passes=False),
        scratch_types=[
            pltpu.VMEM((NB, K), jnp.float32),
            pltpu.VMEM((NB, K), jnp.float32),
            pltpu.VMEM((NB, K), jnp.float32),
            pltpu.VMEM((NB, K), jnp.float32),
            pltpu.VMEM((BPW, G), jnp.int32),
            pltpu.VMEM((BPW, G), jnp.float32),
            pltpu.VMEM((BPW, K), jnp.int32),
            pltpu.VMEM((BPW, K), jnp.int32),
            pltpu.VMEM((3 * L,), jnp.float32),
            pltpu.SemaphoreType.DMA,
            pltpu.SemaphoreType.DMA,
            pltpu.SemaphoreType.DMA,
        ],
    )(scores_t, resid_t,
      heading_class_label, heading_residual_label,
      object_assignment, objectness_label)

    sums = partials.reshape(NW, 3, L).sum(axis=(0, 2))
    denom = sums[2] + 1e-6
    return (sums[0] / denom, sums[1] / denom)


# speedup vs baseline: 1.0237x; 1.0237x over previous
import jax
import jax.numpy as jnp
from jax import lax
from jax.experimental import pallas as pl
from jax.experimental.pallas import tpu as pltpu
from jax.experimental.pallas import tpu_sc as plsc

NB = 12
B = 64
K = 1024
G = 128
NC = 2
NS = 16
NW = NC * NS
BPW = B // NW
L = 16
GROUPS = K // L

_LN2 = 0.6931471805599453
_INV_DELTA = float(NB) / 3.141592653589793

_C = (2.5546730196161803e-07, 0.999967080943859, -0.49928504912250304,
      0.32722571497347896, -0.22316586411879943, 0.13083342798333364,
      -0.05243753706703084, 0.010009289617861138)


def _log_f32(x):
    xi = plsc.bitcast(x, jnp.int32)
    e = (xi >> 23) - 127
    f = plsc.bitcast((xi & 0x007FFFFF) | 0x3F800000, jnp.float32) - 1.0
    f2 = f * f
    f4 = f2 * f2
    q0 = (_C[0] + _C[1] * f) + (_C[2] + _C[3] * f) * f2
    q1 = (_C[4] + _C[5] * f) + (_C[6] + _C[7] * f) * f2
    return e.astype(jnp.float32) * _LN2 + (q0 + q1 * f4)


def _tree(fn, xs):
    xs = list(xs)
    while len(xs) > 1:
        nxt = [fn(xs[i], xs[i + 1]) for i in range(0, len(xs) - 1, 2)]
        if len(xs) % 2:
            nxt.append(xs[-1])
        xs = nxt
    return xs[0]


UNROLL = 4


def _sc_body(scores_hbm, resid_hbm, cls_hbm, rlab_hbm, oa_hbm, obj_hbm,
             out_hbm, scores_v0, resid_v0, scores_v1, resid_v1,
             cls_v, rlab_v, oa_v, obj_v, stage_v, sem0, sem1):
    wid = lax.axis_index("s") * NC + lax.axis_index("c")
    iota16 = lax.iota(jnp.int32, L)
    b0 = wid * BPW

    cp0s = pltpu.async_copy(scores_hbm.at[:, b0], scores_v0, sem0)
    cp0r = pltpu.async_copy(resid_hbm.at[:, b0], resid_v0, sem0)
    pltpu.sync_copy(cls_hbm.at[pl.ds(b0, BPW)], cls_v)
    pltpu.sync_copy(rlab_hbm.at[pl.ds(b0, BPW)], rlab_v)
    pltpu.sync_copy(oa_hbm.at[pl.ds(b0, BPW)], oa_v)
    pltpu.sync_copy(obj_hbm.at[pl.ds(b0, BPW)], obj_v)
    cp1s = pltpu.async_copy(scores_hbm.at[:, b0 + 1], scores_v1, sem1)
    cp1r = pltpu.async_copy(resid_hbm.at[:, b0 + 1], resid_v1, sem1)

    acc = (jnp.zeros((L,), jnp.float32),
           jnp.zeros((L,), jnp.float32),
           jnp.zeros((L,), jnp.float32))

    for b_local in range(BPW):
        scores_v = (scores_v0, scores_v1)[b_local]
        resid_v = (resid_v0, resid_v1)[b_local]
        if b_local == 0:
            cp0s.wait()
            cp0r.wait()
        else:
            cp1s.wait()
            cp1r.wait()

        @plsc.parallel_loop(0, GROUPS, 1, unroll=UNROLL, carry=acc)
        def acc(g, carry):
            acc_ce, acc_hu, acc_obj = carry
            base = g * L
            oa = oa_v[b_local, pl.ds(base, L)]
            obj = obj_v[b_local, pl.ds(base, L)].astype(jnp.float32)
            hcl = plsc.load_gather(cls_v, [jnp.full((L,), b_local, jnp.int32), oa])
            rows = base + iota16
            svals = [scores_v[j, pl.ds(base, L)] for j in range(NB)]
            m = _tree(jnp.maximum, svals)
            se = _tree(jnp.add, [jnp.exp(s - m) for s in svals])
            lse = _log_f32(se) + m
            s_h = plsc.load_gather(scores_v, [hcl, rows])
            ce = lse - s_h
            hrl = plsc.load_gather(
                rlab_v, [jnp.full((L,), b_local, jnp.int32), oa]) * _INV_DELTA
            rn = plsc.load_gather(resid_v, [hcl, rows])
            err = rn - hrl
            ae = jnp.abs(err)
            q = jnp.minimum(ae, 1.0)
            hub = 0.5 * q * q + (ae - q)
            return (acc_ce + ce * obj, acc_hu + hub * obj, acc_obj + obj)

    stage_v[pl.ds(0, L)] = acc[0]
    stage_v[pl.ds(L, L)] = acc[1]
    stage_v[pl.ds(2 * L, L)] = acc[2]
    pltpu.sync_copy(stage_v, out_hbm.at[wid])


@jax.jit
def kernel(heading_class_label, heading_scores, heading_residual_label,
           heading_residuals_normalized, object_assignment, objectness_label):
    scores_t = jnp.transpose(heading_scores, (2, 0, 1))
    resid_t = jnp.transpose(heading_residuals_normalized, (2, 0, 1))

    mesh = plsc.VectorSubcoreMesh(core_axis_name="c", subcore_axis_name="s",
                                  num_cores=NC, num_subcores=NS)
    partials = pl.kernel(
        _sc_body,
        out_type=jax.ShapeDtypeStruct((NW, 3 * L), jnp.float32),
        mesh=mesh,
        compiler_params=pltpu.CompilerParams(needs_layout_passes=False),
        scratch_types=[
            pltpu.VMEM((NB, K), jnp.float32),
            pltpu.VMEM((NB, K), jnp.float32),
            pltpu.VMEM((NB, K), jnp.float32),
            pltpu.VMEM((NB, K), jnp.float32),
            pltpu.VMEM((BPW, G), jnp.int32),
            pltpu.VMEM((BPW, G), jnp.float32),
            pltpu.VMEM((BPW, K), jnp.int32),
            pltpu.VMEM((BPW, K), jnp.int32),
            pltpu.VMEM((3 * L,), jnp.float32),
            pltpu.SemaphoreType.DMA,
            pltpu.SemaphoreType.DMA,
        ],
    )(scores_t, resid_t,
      heading_class_label, heading_residual_label,
      object_assignment, objectness_label)

    sums = partials.reshape(NW, 3, L).sum(axis=(0, 2))
    denom = sums[2] + 1e-6
    return (sums[0] / denom, sums[1] / denom)


# speedup vs baseline: 1.0623x; 1.0377x over previous
import jax
import jax.numpy as jnp
from jax import lax
from jax.experimental import pallas as pl
from jax.experimental.pallas import tpu as pltpu
from jax.experimental.pallas import tpu_sc as plsc

NB = 12
B = 64
K = 1024
G = 128
NC = 2
NS = 16
NW = NC * NS
BPW = B // NW
L = 16
GROUPS = K // L

_LN2 = 0.6931471805599453
_INV_DELTA = float(NB) / 3.141592653589793

_C = (2.5546730196161803e-07, 0.999967080943859, -0.49928504912250304,
      0.32722571497347896, -0.22316586411879943, 0.13083342798333364,
      -0.05243753706703084, 0.010009289617861138)


def _log_f32(x):
    xi = plsc.bitcast(x, jnp.int32)
    e = (xi >> 23) - 127
    f = plsc.bitcast((xi & 0x007FFFFF) | 0x3F800000, jnp.float32) - 1.0
    f2 = f * f
    f4 = f2 * f2
    q0 = (_C[0] + _C[1] * f) + (_C[2] + _C[3] * f) * f2
    q1 = (_C[4] + _C[5] * f) + (_C[6] + _C[7] * f) * f2
    return e.astype(jnp.float32) * _LN2 + (q0 + q1 * f4)


def _tree(fn, xs):
    xs = list(xs)
    while len(xs) > 1:
        nxt = [fn(xs[i], xs[i + 1]) for i in range(0, len(xs) - 1, 2)]
        if len(xs) % 2:
            nxt.append(xs[-1])
        xs = nxt
    return xs[0]


UNROLL = 8


def _sc_body(scores_hbm, resid_hbm, cls_hbm, rlab_hbm, oa_hbm, obj_hbm,
             out_hbm, scores_v0, resid_v0, scores_v1, resid_v1,
             cls_v, rlab_v, oa_v, obj_v, stage_v, sem0, sem1):
    wid = lax.axis_index("s") * NC + lax.axis_index("c")
    iota16 = lax.iota(jnp.int32, L)
    b0 = wid * BPW

    cp0s = pltpu.async_copy(scores_hbm.at[:, b0], scores_v0, sem0)
    cp0r = pltpu.async_copy(resid_hbm.at[:, b0], resid_v0, sem0)
    pltpu.sync_copy(cls_hbm.at[pl.ds(b0, BPW)], cls_v)
    pltpu.sync_copy(rlab_hbm.at[pl.ds(b0, BPW)], rlab_v)
    pltpu.sync_copy(oa_hbm.at[pl.ds(b0, BPW)], oa_v)
    pltpu.sync_copy(obj_hbm.at[pl.ds(b0, BPW)], obj_v)
    cp1s = pltpu.async_copy(scores_hbm.at[:, b0 + 1], scores_v1, sem1)
    cp1r = pltpu.async_copy(resid_hbm.at[:, b0 + 1], resid_v1, sem1)

    acc = (jnp.zeros((L,), jnp.float32),
           jnp.zeros((L,), jnp.float32),
           jnp.zeros((L,), jnp.float32))

    for b_local in range(BPW):
        scores_v = (scores_v0, scores_v1)[b_local]
        resid_v = (resid_v0, resid_v1)[b_local]
        if b_local == 0:
            cp0s.wait()
            cp0r.wait()
        else:
            cp1s.wait()
            cp1r.wait()

        @plsc.parallel_loop(0, GROUPS, 1, unroll=UNROLL, carry=acc)
        def acc(g, carry):
            acc_ce, acc_hu, acc_obj = carry
            base = g * L
            oa = oa_v[b_local, pl.ds(base, L)]
            obj = obj_v[b_local, pl.ds(base, L)].astype(jnp.float32)
            hcl = plsc.load_gather(cls_v, [jnp.full((L,), b_local, jnp.int32), oa])
            rows = base + iota16
            svals = [scores_v[j, pl.ds(base, L)] for j in range(NB)]
            m = _tree(jnp.maximum, svals)
            se = _tree(jnp.add, [jnp.exp(s - m) for s in svals])
            lse = _log_f32(se) + m
            s_h = plsc.load_gather(scores_v, [hcl, rows])
            ce = lse - s_h
            hrl = plsc.load_gather(
                rlab_v, [jnp.full((L,), b_local, jnp.int32), oa]) * _INV_DELTA
            rn = plsc.load_gather(resid_v, [hcl, rows])
            err = rn - hrl
            ae = jnp.abs(err)
            q = jnp.minimum(ae, 1.0)
            hub = 0.5 * q * q + (ae - q)
            return (acc_ce + ce * obj, acc_hu + hub * obj, acc_obj + obj)

    stage_v[pl.ds(0, L)] = acc[0]
    stage_v[pl.ds(L, L)] = acc[1]
    stage_v[pl.ds(2 * L, L)] = acc[2]
    pltpu.sync_copy(stage_v, out_hbm.at[wid])


@jax.jit
def kernel(heading_class_label, heading_scores, heading_residual_label,
           heading_residuals_normalized, object_assignment, objectness_label):
    scores_t = jnp.transpose(heading_scores, (2, 0, 1))
    resid_t = jnp.transpose(heading_residuals_normalized, (2, 0, 1))

    mesh = plsc.VectorSubcoreMesh(core_axis_name="c", subcore_axis_name="s",
                                  num_cores=NC, num_subcores=NS)
    partials = pl.kernel(
        _sc_body,
        out_type=jax.ShapeDtypeStruct((NW, 3 * L), jnp.float32),
        mesh=mesh,
        compiler_params=pltpu.CompilerParams(needs_layout_passes=False),
        scratch_types=[
            pltpu.VMEM((NB, K), jnp.float32),
            pltpu.VMEM((NB, K), jnp.float32),
            pltpu.VMEM((NB, K), jnp.float32),
            pltpu.VMEM((NB, K), jnp.float32),
            pltpu.VMEM((BPW, G), jnp.int32),
            pltpu.VMEM((BPW, G), jnp.float32),
            pltpu.VMEM((BPW, K), jnp.int32),
            pltpu.VMEM((BPW, K), jnp.int32),
            pltpu.VMEM((3 * L,), jnp.float32),
            pltpu.SemaphoreType.DMA,
            pltpu.SemaphoreType.DMA,
        ],
    )(scores_t, resid_t,
      heading_class_label, heading_residual_label,
      object_assignment, objectness_label)

    sums = partials.reshape(NW, 3, L).sum(axis=(0, 2))
    denom = sums[2] + 1e-6
    return (sums[0] / denom, sums[1] / denom)
